# baseline (speedup 1.0000x reference)
import functools

import jax
import jax.numpy as jnp
from jax import lax
from jax.experimental import pallas as pl
from jax.experimental.pallas import tpu as pltpu
from jax.experimental.pallas import tpu_sc as plsc

MCFG_EPS = 0.0005

N_CLOTH = 32768
N_OBS_VERTS = 16384
N_OBS_FACES = 16384

NC = 2
NS = 16
NW = NC * NS
L = 16


_FP = N_OBS_FACES // NW


def _faces_body(vx_h, vy_h, vz_h, f0_h, f1_h, f2_h,
                cent_h, norm_h,
                vx, vy, vz, f0, f1, f2, c0, c1, c2, c3, c4, c5, c6):
    wid = lax.axis_index("s") * NC + lax.axis_index("c")
    base = wid * _FP
    pltpu.sync_copy(vx_h, vx)
    pltpu.sync_copy(vy_h, vy)
    pltpu.sync_copy(vz_h, vz)
    pltpu.sync_copy(f0_h.at[pl.ds(base, _FP)], f0)
    pltpu.sync_copy(f1_h.at[pl.ds(base, _FP)], f1)
    pltpu.sync_copy(f2_h.at[pl.ds(base, _FP)], f2)
    for i in range(_FP // L):
        sl = pl.ds(i * L, L)
        i0 = f0[sl]
        i1 = f1[sl]
        i2 = f2[sl]
        x0 = plsc.load_gather(vx, [i0]); y0 = plsc.load_gather(vy, [i0]); z0 = plsc.load_gather(vz, [i0])
        x1 = plsc.load_gather(vx, [i1]); y1 = plsc.load_gather(vy, [i1]); z1 = plsc.load_gather(vz, [i1])
        x2 = plsc.load_gather(vx, [i2]); y2 = plsc.load_gather(vy, [i2]); z2 = plsc.load_gather(vz, [i2])
        cxv = (x0 + x1 + x2) / 3.0
        cyv = (y0 + y1 + y2) / 3.0
        czv = (z0 + z1 + z2) / 3.0
        c0[sl] = cxv
        c1[sl] = cyv
        c2[sl] = czv
        c6[sl] = cxv * cxv + cyv * cyv + czv * czv
        ux = x1 - x0; uy = y1 - y0; uz = z1 - z0
        wx = x2 - x0; wy = y2 - y0; wz = z2 - z0
        c3[sl] = uy * wz - uz * wy
        c4[sl] = uz * wx - ux * wz
        c5[sl] = ux * wy - uy * wx
    F = N_OBS_FACES
    pltpu.sync_copy(c0, cent_h.at[pl.ds(base, _FP)])
    pltpu.sync_copy(c1, cent_h.at[pl.ds(F + base, _FP)])
    pltpu.sync_copy(c2, cent_h.at[pl.ds(2 * F + base, _FP)])
    pltpu.sync_copy(c6, cent_h.at[pl.ds(3 * F + base, _FP)])
    pltpu.sync_copy(c3, norm_h.at[pl.ds(base, _FP)])
    pltpu.sync_copy(c4, norm_h.at[pl.ds(F + base, _FP)])
    pltpu.sync_copy(c5, norm_h.at[pl.ds(2 * F + base, _FP)])


@functools.lru_cache(maxsize=None)
def _faces_kernel_build():
    return functools.partial(
        pl.kernel,
        out_type=[jax.ShapeDtypeStruct((4 * N_OBS_FACES,), jnp.float32),
                  jax.ShapeDtypeStruct((3 * N_OBS_FACES,), jnp.float32)],
        mesh=plsc.VectorSubcoreMesh(core_axis_name="c", subcore_axis_name="s",
                                    num_cores=NC, num_subcores=NS),
        compiler_params=pltpu.CompilerParams(needs_layout_passes=False),
        scratch_types=(
            [pltpu.VMEM((N_OBS_VERTS,), jnp.float32)] * 3
            + [pltpu.VMEM((_FP,), jnp.int32)] * 3
            + [pltpu.VMEM((_FP,), jnp.float32)] * 7
        ),
    )(_faces_body)


_BQ = 512
_W = 128
_NCHUNK = N_OBS_FACES // _W


def _knn_body(q_ref, cent_ref, norm_ref,
              idx_ref, nhat_ref):
    pid = pl.program_id(0)

    @pl.when(pid == 0)
    def _():
        nx = norm_ref[0:1, :]
        ny = norm_ref[1:2, :]
        nz = norm_ref[2:3, :]
        inv = 1.0 / (jnp.sqrt(nx * nx + ny * ny + nz * nz) + 1e-12)
        nhat_ref[0:1, :] = nx * inv
        nhat_ref[1:2, :] = ny * inv
        nhat_ref[2:3, :] = nz * inv

    q = q_ref[...]
    qsq = jnp.sum(q * q, axis=1, keepdims=True)
    qm2 = q * -2.0

    def dist(c):
        sl = pl.ds(c * _W, _W)
        k3 = cent_ref[0:3, sl]
        ksq = cent_ref[3:4, sl]
        t2 = jax.lax.dot_general(
            qm2, k3, (((1,), (0,)), ((), ())),
            preferred_element_type=jnp.float32)
        return (qsq + t2) + ksq

    def merge(a, b):
        ma, ia = a
        mb, ib = b
        return jnp.minimum(ma, mb), jnp.where(mb < ma, ib, ia)

    nodes = [(dist(c), jnp.int32(c)) for c in range(_NCHUNK)]
    while len(nodes) > 1:
        nodes = [merge(nodes[i], nodes[i + 1])
                 for i in range(0, len(nodes), 2)]
    m, ci = nodes[0]

    lane = lax.broadcasted_iota(jnp.int32, (_BQ, _W), 1)
    gidx = ci * _W + lane
    rowmin = jnp.min(m, axis=1, keepdims=True)
    cand = jnp.where(m == rowmin, gidx, jnp.int32(2**31 - 1))
    idx_ref[...] = jnp.min(cand, axis=1, keepdims=True)


def _knn_call(target_pos, cents, norms):
    grid = N_CLOTH // _BQ
    kspec = pl.BlockSpec((3, N_OBS_FACES), lambda i: (0, 0))
    return pl.pallas_call(
        _knn_body,
        grid=(grid,),
        in_specs=[
            pl.BlockSpec((_BQ, 3), lambda i: (i, 0)),
            pl.BlockSpec((4, N_OBS_FACES), lambda i: (0, 0)),
            kspec,
        ],
        out_specs=[
            pl.BlockSpec((_BQ, 1), lambda i: (i, 0)),
            kspec,
        ],
        out_shape=[
            jax.ShapeDtypeStruct((N_CLOTH, 1), jnp.int32),
            jax.ShapeDtypeStruct((3, N_OBS_FACES), jnp.float32),
        ],
    )(target_pos, cents, norms)


_QP = N_CLOTH // NW


def _gather_body(cent_h, nhat_h, idx_h, gat_h,
                 tcx, tcy, tcz, tnx, tny, tnz, qidx,
                 g0, g1, g2, g3, g4, g5):
    wid = lax.axis_index("s") * NC + lax.axis_index("c")
    base = wid * _QP
    F = N_OBS_FACES
    N = N_CLOTH
    pltpu.sync_copy(cent_h.at[pl.ds(0, F)], tcx)
    pltpu.sync_copy(cent_h.at[pl.ds(F, F)], tcy)
    pltpu.sync_copy(cent_h.at[pl.ds(2 * F, F)], tcz)
    pltpu.sync_copy(nhat_h.at[pl.ds(0, F)], tnx)
    pltpu.sync_copy(nhat_h.at[pl.ds(F, F)], tny)
    pltpu.sync_copy(nhat_h.at[pl.ds(2 * F, F)], tnz)
    pltpu.sync_copy(idx_h.at[pl.ds(base, _QP)], qidx)
    for i in range(_QP // L):
        sl = pl.ds(i * L, L)
        ii = qidx[sl]
        g0[sl] = plsc.load_gather(tcx, [ii])
        g1[sl] = plsc.load_gather(tcy, [ii])
        g2[sl] = plsc.load_gather(tcz, [ii])
        g3[sl] = plsc.load_gather(tnx, [ii])
        g4[sl] = plsc.load_gather(tny, [ii])
        g5[sl] = plsc.load_gather(tnz, [ii])
    pltpu.sync_copy(g0, gat_h.at[pl.ds(base, _QP)])
    pltpu.sync_copy(g1, gat_h.at[pl.ds(N + base, _QP)])
    pltpu.sync_copy(g2, gat_h.at[pl.ds(2 * N + base, _QP)])
    pltpu.sync_copy(g3, gat_h.at[pl.ds(3 * N + base, _QP)])
    pltpu.sync_copy(g4, gat_h.at[pl.ds(4 * N + base, _QP)])
    pltpu.sync_copy(g5, gat_h.at[pl.ds(5 * N + base, _QP)])


@functools.lru_cache(maxsize=None)
def _gather_kernel_build():
    return functools.partial(
        pl.kernel,
        out_type=jax.ShapeDtypeStruct((6 * N_CLOTH,), jnp.float32),
        mesh=plsc.VectorSubcoreMesh(core_axis_name="c", subcore_axis_name="s",
                                    num_cores=NC, num_subcores=NS),
        compiler_params=pltpu.CompilerParams(needs_layout_passes=False),
        scratch_types=(
            [pltpu.VMEM((N_OBS_FACES,), jnp.float32)] * 6
            + [pltpu.VMEM((_QP,), jnp.int32)]
            + [pltpu.VMEM((_QP,), jnp.float32)] * 6
        ),
    )(_gather_body)


def _loss_body(pt_ref, tt_ref, bm_ref, gat_ref, out_ref):
    px = pt_ref[0:1, :]; py = pt_ref[1:2, :]; pz = pt_ref[2:3, :]
    tx = tt_ref[0:1, :]; ty = tt_ref[1:2, :]; tz = tt_ref[2:3, :]
    dx = px - tx; dy = py - ty; dz = pz - tz
    loss = (dx * dx + dy * dy + dz * dz) * bm_ref[...]
    dist = ((tx - gat_ref[0:1, :]) * gat_ref[3:4, :]
            + (ty - gat_ref[1:2, :]) * gat_ref[4:5, :]
            + (tz - gat_ref[2:3, :]) * gat_ref[5:6, :])
    pen = jnp.where(dist > MCFG_EPS, 1.0, 0.0)
    total = jnp.sum(loss * pen, axis=1, keepdims=True)
    out_ref[...] = total * (1.0 / N_CLOTH)


def _loss_call(predT, targetT, bmask, gat):
    return pl.pallas_call(
        _loss_body,
        in_specs=[
            pl.BlockSpec((3, N_CLOTH), lambda: (0, 0)),
            pl.BlockSpec((3, N_CLOTH), lambda: (0, 0)),
            pl.BlockSpec((1, N_CLOTH), lambda: (0, 0)),
            pl.BlockSpec((6, N_CLOTH), lambda: (0, 0)),
        ],
        out_specs=pl.BlockSpec((1, 1), lambda: (0, 0)),
        out_shape=jax.ShapeDtypeStruct((1, 1), jnp.float32),
    )(predT, targetT, bmask, gat)


def kernel(cloth_pred_pos, cloth_target_pos, cloth_boundary_mask,
           obstacle_target_pos, obstacle_faces):
    faces = obstacle_faces.astype(jnp.int32)
    vx = obstacle_target_pos[:, 0]
    vy = obstacle_target_pos[:, 1]
    vz = obstacle_target_pos[:, 2]
    centf, normf = _faces_kernel_build()(
        vx, vy, vz, faces[0], faces[1], faces[2])

    nn_idx, nhat = _knn_call(
        cloth_target_pos,
        centf.reshape(4, N_OBS_FACES),
        normf.reshape(3, N_OBS_FACES))

    gatf = _gather_kernel_build()(
        centf, nhat.reshape(-1), nn_idx.reshape(-1))

    out = _loss_call(
        cloth_pred_pos.T, cloth_target_pos.T,
        cloth_boundary_mask.reshape(1, -1),
        gatf.reshape(6, N_CLOTH))
    return out[0, 0]


# speedup vs baseline: 1.0769x; 1.0769x over previous
import functools

import jax
import jax.numpy as jnp
from jax import lax
from jax.experimental import pallas as pl
from jax.experimental.pallas import tpu as pltpu
from jax.experimental.pallas import tpu_sc as plsc

MCFG_EPS = 0.0005

N_CLOTH = 32768
N_OBS_VERTS = 16384
N_OBS_FACES = 16384

NC = 2
NS = 16
NW = NC * NS
L = 16


_FP = N_OBS_FACES // NW


def _faces_body(vx_h, vy_h, vz_h, f0_h, f1_h, f2_h,
                cent_h, norm_h,
                vx, vy, vz, f0, f1, f2, c0, c1, c2, c3, c4, c5, c6):
    wid = lax.axis_index("s") * NC + lax.axis_index("c")
    base = wid * _FP
    pltpu.sync_copy(vx_h, vx)
    pltpu.sync_copy(vy_h, vy)
    pltpu.sync_copy(vz_h, vz)
    pltpu.sync_copy(f0_h.at[pl.ds(base, _FP)], f0)
    pltpu.sync_copy(f1_h.at[pl.ds(base, _FP)], f1)
    pltpu.sync_copy(f2_h.at[pl.ds(base, _FP)], f2)
    for i in range(_FP // L):
        sl = pl.ds(i * L, L)
        i0 = f0[sl]
        i1 = f1[sl]
        i2 = f2[sl]
        x0 = plsc.load_gather(vx, [i0]); y0 = plsc.load_gather(vy, [i0]); z0 = plsc.load_gather(vz, [i0])
        x1 = plsc.load_gather(vx, [i1]); y1 = plsc.load_gather(vy, [i1]); z1 = plsc.load_gather(vz, [i1])
        x2 = plsc.load_gather(vx, [i2]); y2 = plsc.load_gather(vy, [i2]); z2 = plsc.load_gather(vz, [i2])
        cxv = (x0 + x1 + x2) / 3.0
        cyv = (y0 + y1 + y2) / 3.0
        czv = (z0 + z1 + z2) / 3.0
        c0[sl] = cxv
        c1[sl] = cyv
        c2[sl] = czv
        c6[sl] = cxv * cxv + cyv * cyv + czv * czv
        ux = x1 - x0; uy = y1 - y0; uz = z1 - z0
        wx = x2 - x0; wy = y2 - y0; wz = z2 - z0
        c3[sl] = uy * wz - uz * wy
        c4[sl] = uz * wx - ux * wz
        c5[sl] = ux * wy - uy * wx
    F = N_OBS_FACES
    pltpu.sync_copy(c0, cent_h.at[pl.ds(base, _FP)])
    pltpu.sync_copy(c1, cent_h.at[pl.ds(F + base, _FP)])
    pltpu.sync_copy(c2, cent_h.at[pl.ds(2 * F + base, _FP)])
    pltpu.sync_copy(c6, cent_h.at[pl.ds(3 * F + base, _FP)])
    pltpu.sync_copy(c3, norm_h.at[pl.ds(base, _FP)])
    pltpu.sync_copy(c4, norm_h.at[pl.ds(F + base, _FP)])
    pltpu.sync_copy(c5, norm_h.at[pl.ds(2 * F + base, _FP)])


@functools.lru_cache(maxsize=None)
def _faces_kernel_build():
    return functools.partial(
        pl.kernel,
        out_type=[jax.ShapeDtypeStruct((4 * N_OBS_FACES,), jnp.float32),
                  jax.ShapeDtypeStruct((3 * N_OBS_FACES,), jnp.float32)],
        mesh=plsc.VectorSubcoreMesh(core_axis_name="c", subcore_axis_name="s",
                                    num_cores=NC, num_subcores=NS),
        compiler_params=pltpu.CompilerParams(needs_layout_passes=False),
        scratch_types=(
            [pltpu.VMEM((N_OBS_VERTS,), jnp.float32)] * 3
            + [pltpu.VMEM((_FP,), jnp.int32)] * 3
            + [pltpu.VMEM((_FP,), jnp.float32)] * 7
        ),
    )(_faces_body)


_BQ = 1024
_W = 256
_NCHUNK = N_OBS_FACES // _W


def _knn_body(q_ref, cent_ref, norm_ref,
              idx_ref, nhat_ref):
    pid = pl.program_id(0)

    @pl.when(pid == 0)
    def _():
        nx = norm_ref[0:1, :]
        ny = norm_ref[1:2, :]
        nz = norm_ref[2:3, :]
        inv = 1.0 / (jnp.sqrt(nx * nx + ny * ny + nz * nz) + 1e-12)
        nhat_ref[0:1, :] = nx * inv
        nhat_ref[1:2, :] = ny * inv
        nhat_ref[2:3, :] = nz * inv

    q = q_ref[...]
    qsq = jnp.sum(q * q, axis=1, keepdims=True)
    qm2 = q * -2.0

    def dist(c):
        sl = pl.ds(c * _W, _W)
        k3 = cent_ref[0:3, sl]
        ksq = cent_ref[3:4, sl]
        t2 = jax.lax.dot_general(
            qm2, k3, (((1,), (0,)), ((), ())),
            preferred_element_type=jnp.float32)
        return (qsq + t2) + ksq

    def merge(a, b):
        ma, ia = a
        mb, ib = b
        return jnp.minimum(ma, mb), jnp.where(mb < ma, ib, ia)

    nodes = [(dist(c), jnp.int32(c)) for c in range(_NCHUNK)]
    while len(nodes) > 1:
        nodes = [merge(nodes[i], nodes[i + 1])
                 for i in range(0, len(nodes), 2)]
    m, ci = nodes[0]

    lane = lax.broadcasted_iota(jnp.int32, (_BQ, _W), 1)
    gidx = ci * _W + lane
    rowmin = jnp.min(m, axis=1, keepdims=True)
    cand = jnp.where(m == rowmin, gidx, jnp.int32(2**31 - 1))
    idx_ref[...] = jnp.min(cand, axis=1, keepdims=True)


def _knn_call(target_pos, cents, norms):
    grid = N_CLOTH // _BQ
    kspec = pl.BlockSpec((3, N_OBS_FACES), lambda i: (0, 0))
    return pl.pallas_call(
        _knn_body,
        grid=(grid,),
        in_specs=[
            pl.BlockSpec((_BQ, 3), lambda i: (i, 0)),
            pl.BlockSpec((4, N_OBS_FACES), lambda i: (0, 0)),
            kspec,
        ],
        out_specs=[
            pl.BlockSpec((_BQ, 1), lambda i: (i, 0)),
            kspec,
        ],
        out_shape=[
            jax.ShapeDtypeStruct((N_CLOTH, 1), jnp.int32),
            jax.ShapeDtypeStruct((3, N_OBS_FACES), jnp.float32),
        ],
    )(target_pos, cents, norms)


_QP = N_CLOTH // NW


def _gather_body(cent_h, nhat_h, idx_h, gat_h,
                 tcx, tcy, tcz, tnx, tny, tnz, qidx,
                 g0, g1, g2, g3, g4, g5):
    wid = lax.axis_index("s") * NC + lax.axis_index("c")
    base = wid * _QP
    F = N_OBS_FACES
    N = N_CLOTH
    pltpu.sync_copy(cent_h.at[pl.ds(0, F)], tcx)
    pltpu.sync_copy(cent_h.at[pl.ds(F, F)], tcy)
    pltpu.sync_copy(cent_h.at[pl.ds(2 * F, F)], tcz)
    pltpu.sync_copy(nhat_h.at[pl.ds(0, F)], tnx)
    pltpu.sync_copy(nhat_h.at[pl.ds(F, F)], tny)
    pltpu.sync_copy(nhat_h.at[pl.ds(2 * F, F)], tnz)
    pltpu.sync_copy(idx_h.at[pl.ds(base, _QP)], qidx)
    for i in range(_QP // L):
        sl = pl.ds(i * L, L)
        ii = qidx[sl]
        g0[sl] = plsc.load_gather(tcx, [ii])
        g1[sl] = plsc.load_gather(tcy, [ii])
        g2[sl] = plsc.load_gather(tcz, [ii])
        g3[sl] = plsc.load_gather(tnx, [ii])
        g4[sl] = plsc.load_gather(tny, [ii])
        g5[sl] = plsc.load_gather(tnz, [ii])
    pltpu.sync_copy(g0, gat_h.at[pl.ds(base, _QP)])
    pltpu.sync_copy(g1, gat_h.at[pl.ds(N + base, _QP)])
    pltpu.sync_copy(g2, gat_h.at[pl.ds(2 * N + base, _QP)])
    pltpu.sync_copy(g3, gat_h.at[pl.ds(3 * N + base, _QP)])
    pltpu.sync_copy(g4, gat_h.at[pl.ds(4 * N + base, _QP)])
    pltpu.sync_copy(g5, gat_h.at[pl.ds(5 * N + base, _QP)])


@functools.lru_cache(maxsize=None)
def _gather_kernel_build():
    return functools.partial(
        pl.kernel,
        out_type=jax.ShapeDtypeStruct((6 * N_CLOTH,), jnp.float32),
        mesh=plsc.VectorSubcoreMesh(core_axis_name="c", subcore_axis_name="s",
                                    num_cores=NC, num_subcores=NS),
        compiler_params=pltpu.CompilerParams(needs_layout_passes=False),
        scratch_types=(
            [pltpu.VMEM((N_OBS_FACES,), jnp.float32)] * 6
            + [pltpu.VMEM((_QP,), jnp.int32)]
            + [pltpu.VMEM((_QP,), jnp.float32)] * 6
        ),
    )(_gather_body)


def _loss_body(pt_ref, tt_ref, bm_ref, gat_ref, out_ref):
    px = pt_ref[0:1, :]; py = pt_ref[1:2, :]; pz = pt_ref[2:3, :]
    tx = tt_ref[0:1, :]; ty = tt_ref[1:2, :]; tz = tt_ref[2:3, :]
    dx = px - tx; dy = py - ty; dz = pz - tz
    loss = (dx * dx + dy * dy + dz * dz) * bm_ref[...]
    dist = ((tx - gat_ref[0:1, :]) * gat_ref[3:4, :]
            + (ty - gat_ref[1:2, :]) * gat_ref[4:5, :]
            + (tz - gat_ref[2:3, :]) * gat_ref[5:6, :])
    pen = jnp.where(dist > MCFG_EPS, 1.0, 0.0)
    total = jnp.sum(loss * pen, axis=1, keepdims=True)
    out_ref[...] = total * (1.0 / N_CLOTH)


def _loss_call(predT, targetT, bmask, gat):
    return pl.pallas_call(
        _loss_body,
        in_specs=[
            pl.BlockSpec((3, N_CLOTH), lambda: (0, 0)),
            pl.BlockSpec((3, N_CLOTH), lambda: (0, 0)),
            pl.BlockSpec((1, N_CLOTH), lambda: (0, 0)),
            pl.BlockSpec((6, N_CLOTH), lambda: (0, 0)),
        ],
        out_specs=pl.BlockSpec((1, 1), lambda: (0, 0)),
        out_shape=jax.ShapeDtypeStruct((1, 1), jnp.float32),
    )(predT, targetT, bmask, gat)


def kernel(cloth_pred_pos, cloth_target_pos, cloth_boundary_mask,
           obstacle_target_pos, obstacle_faces):
    faces = obstacle_faces.astype(jnp.int32)
    vx = obstacle_target_pos[:, 0]
    vy = obstacle_target_pos[:, 1]
    vz = obstacle_target_pos[:, 2]
    centf, normf = _faces_kernel_build()(
        vx, vy, vz, faces[0], faces[1], faces[2])

    nn_idx, nhat = _knn_call(
        cloth_target_pos,
        centf.reshape(4, N_OBS_FACES),
        normf.reshape(3, N_OBS_FACES))

    gatf = _gather_kernel_build()(
        centf, nhat.reshape(-1), nn_idx.reshape(-1))

    out = _loss_call(
        cloth_pred_pos.T, cloth_target_pos.T,
        cloth_boundary_mask.reshape(1, -1),
        gatf.reshape(6, N_CLOTH))
    return out[0, 0]


# speedup vs baseline: 1.0792x; 1.0021x over previous
import functools

import jax
import jax.numpy as jnp
from jax import lax
from jax.experimental import pallas as pl
from jax.experimental.pallas import tpu as pltpu
from jax.experimental.pallas import tpu_sc as plsc

MCFG_EPS = 0.0005

N_CLOTH = 32768
N_OBS_VERTS = 16384
N_OBS_FACES = 16384

NC = 2
NS = 16
NW = NC * NS
L = 16


_FP = N_OBS_FACES // NW


def _faces_body(vx_h, vy_h, vz_h, f0_h, f1_h, f2_h,
                cent_h, norm_h,
                vx, vy, vz, f0, f1, f2, c0, c1, c2, c3, c4, c5, c6):
    wid = lax.axis_index("s") * NC + lax.axis_index("c")
    base = wid * _FP
    pltpu.sync_copy(vx_h, vx)
    pltpu.sync_copy(vy_h, vy)
    pltpu.sync_copy(vz_h, vz)
    pltpu.sync_copy(f0_h.at[pl.ds(base, _FP)], f0)
    pltpu.sync_copy(f1_h.at[pl.ds(base, _FP)], f1)
    pltpu.sync_copy(f2_h.at[pl.ds(base, _FP)], f2)
    for i in range(_FP // L):
        sl = pl.ds(i * L, L)
        i0 = f0[sl]
        i1 = f1[sl]
        i2 = f2[sl]
        x0 = plsc.load_gather(vx, [i0]); y0 = plsc.load_gather(vy, [i0]); z0 = plsc.load_gather(vz, [i0])
        x1 = plsc.load_gather(vx, [i1]); y1 = plsc.load_gather(vy, [i1]); z1 = plsc.load_gather(vz, [i1])
        x2 = plsc.load_gather(vx, [i2]); y2 = plsc.load_gather(vy, [i2]); z2 = plsc.load_gather(vz, [i2])
        cxv = (x0 + x1 + x2) / 3.0
        cyv = (y0 + y1 + y2) / 3.0
        czv = (z0 + z1 + z2) / 3.0
        c0[sl] = cxv
        c1[sl] = cyv
        c2[sl] = czv
        c6[sl] = cxv * cxv + cyv * cyv + czv * czv
        ux = x1 - x0; uy = y1 - y0; uz = z1 - z0
        wx = x2 - x0; wy = y2 - y0; wz = z2 - z0
        c3[sl] = uy * wz - uz * wy
        c4[sl] = uz * wx - ux * wz
        c5[sl] = ux * wy - uy * wx
    F = N_OBS_FACES
    pltpu.sync_copy(c0, cent_h.at[pl.ds(base, _FP)])
    pltpu.sync_copy(c1, cent_h.at[pl.ds(F + base, _FP)])
    pltpu.sync_copy(c2, cent_h.at[pl.ds(2 * F + base, _FP)])
    pltpu.sync_copy(c6, cent_h.at[pl.ds(3 * F + base, _FP)])
    pltpu.sync_copy(c3, norm_h.at[pl.ds(base, _FP)])
    pltpu.sync_copy(c4, norm_h.at[pl.ds(F + base, _FP)])
    pltpu.sync_copy(c5, norm_h.at[pl.ds(2 * F + base, _FP)])


@functools.lru_cache(maxsize=None)
def _faces_kernel_build():
    return functools.partial(
        pl.kernel,
        out_type=[jax.ShapeDtypeStruct((4 * N_OBS_FACES,), jnp.float32),
                  jax.ShapeDtypeStruct((3 * N_OBS_FACES,), jnp.float32)],
        mesh=plsc.VectorSubcoreMesh(core_axis_name="c", subcore_axis_name="s",
                                    num_cores=NC, num_subcores=NS),
        compiler_params=pltpu.CompilerParams(needs_layout_passes=False),
        scratch_types=(
            [pltpu.VMEM((N_OBS_VERTS,), jnp.float32)] * 3
            + [pltpu.VMEM((_FP,), jnp.int32)] * 3
            + [pltpu.VMEM((_FP,), jnp.float32)] * 7
        ),
    )(_faces_body)


_BQ = 2048
_W = 256
_NCHUNK = N_OBS_FACES // _W


def _knn_body(q_ref, cent_ref, norm_ref,
              idx_ref, nhat_ref):
    pid = pl.program_id(0)

    @pl.when(pid == 0)
    def _():
        nx = norm_ref[0:1, :]
        ny = norm_ref[1:2, :]
        nz = norm_ref[2:3, :]
        inv = 1.0 / (jnp.sqrt(nx * nx + ny * ny + nz * nz) + 1e-12)
        nhat_ref[0:1, :] = nx * inv
        nhat_ref[1:2, :] = ny * inv
        nhat_ref[2:3, :] = nz * inv

    q = q_ref[...]
    qsq = jnp.sum(q * q, axis=1, keepdims=True)
    qm2 = q * -2.0

    def dist(c):
        sl = pl.ds(c * _W, _W)
        k3 = cent_ref[0:3, sl]
        ksq = cent_ref[3:4, sl]
        t2 = jax.lax.dot_general(
            qm2, k3, (((1,), (0,)), ((), ())),
            preferred_element_type=jnp.float32)
        return (qsq + t2) + ksq

    def merge(a, b):
        ma, ia = a
        mb, ib = b
        return jnp.minimum(ma, mb), jnp.where(mb < ma, ib, ia)

    nodes = [(dist(c), jnp.int32(c)) for c in range(_NCHUNK)]
    while len(nodes) > 1:
        nodes = [merge(nodes[i], nodes[i + 1])
                 for i in range(0, len(nodes), 2)]
    m, ci = nodes[0]

    lane = lax.broadcasted_iota(jnp.int32, (_BQ, _W), 1)
    gidx = ci * _W + lane
    rowmin = jnp.min(m, axis=1, keepdims=True)
    cand = jnp.where(m == rowmin, gidx, jnp.int32(2**31 - 1))
    idx_ref[...] = jnp.min(cand, axis=1, keepdims=True)


def _knn_call(target_pos, cents, norms):
    grid = N_CLOTH // _BQ
    kspec = pl.BlockSpec((3, N_OBS_FACES), lambda i: (0, 0))
    return pl.pallas_call(
        _knn_body,
        grid=(grid,),
        in_specs=[
            pl.BlockSpec((_BQ, 3), lambda i: (i, 0)),
            pl.BlockSpec((4, N_OBS_FACES), lambda i: (0, 0)),
            kspec,
        ],
        out_specs=[
            pl.BlockSpec((_BQ, 1), lambda i: (i, 0)),
            kspec,
        ],
        out_shape=[
            jax.ShapeDtypeStruct((N_CLOTH, 1), jnp.int32),
            jax.ShapeDtypeStruct((3, N_OBS_FACES), jnp.float32),
        ],
    )(target_pos, cents, norms)


_QP = N_CLOTH // NW


def _gather_body(cent_h, nhat_h, idx_h, gat_h,
                 tcx, tcy, tcz, tnx, tny, tnz, qidx,
                 g0, g1, g2, g3, g4, g5):
    wid = lax.axis_index("s") * NC + lax.axis_index("c")
    base = wid * _QP
    F = N_OBS_FACES
    N = N_CLOTH
    pltpu.sync_copy(cent_h.at[pl.ds(0, F)], tcx)
    pltpu.sync_copy(cent_h.at[pl.ds(F, F)], tcy)
    pltpu.sync_copy(cent_h.at[pl.ds(2 * F, F)], tcz)
    pltpu.sync_copy(nhat_h.at[pl.ds(0, F)], tnx)
    pltpu.sync_copy(nhat_h.at[pl.ds(F, F)], tny)
    pltpu.sync_copy(nhat_h.at[pl.ds(2 * F, F)], tnz)
    pltpu.sync_copy(idx_h.at[pl.ds(base, _QP)], qidx)
    for i in range(_QP // L):
        sl = pl.ds(i * L, L)
        ii = qidx[sl]
        g0[sl] = plsc.load_gather(tcx, [ii])
        g1[sl] = plsc.load_gather(tcy, [ii])
        g2[sl] = plsc.load_gather(tcz, [ii])
        g3[sl] = plsc.load_gather(tnx, [ii])
        g4[sl] = plsc.load_gather(tny, [ii])
        g5[sl] = plsc.load_gather(tnz, [ii])
    pltpu.sync_copy(g0, gat_h.at[pl.ds(base, _QP)])
    pltpu.sync_copy(g1, gat_h.at[pl.ds(N + base, _QP)])
    pltpu.sync_copy(g2, gat_h.at[pl.ds(2 * N + base, _QP)])
    pltpu.sync_copy(g3, gat_h.at[pl.ds(3 * N + base, _QP)])
    pltpu.sync_copy(g4, gat_h.at[pl.ds(4 * N + base, _QP)])
    pltpu.sync_copy(g5, gat_h.at[pl.ds(5 * N + base, _QP)])


@functools.lru_cache(maxsize=None)
def _gather_kernel_build():
    return functools.partial(
        pl.kernel,
        out_type=jax.ShapeDtypeStruct((6 * N_CLOTH,), jnp.float32),
        mesh=plsc.VectorSubcoreMesh(core_axis_name="c", subcore_axis_name="s",
                                    num_cores=NC, num_subcores=NS),
        compiler_params=pltpu.CompilerParams(needs_layout_passes=False),
        scratch_types=(
            [pltpu.VMEM((N_OBS_FACES,), jnp.float32)] * 6
            + [pltpu.VMEM((_QP,), jnp.int32)]
            + [pltpu.VMEM((_QP,), jnp.float32)] * 6
        ),
    )(_gather_body)


def _loss_body(pt_ref, tt_ref, bm_ref, gat_ref, out_ref):
    px = pt_ref[0:1, :]; py = pt_ref[1:2, :]; pz = pt_ref[2:3, :]
    tx = tt_ref[0:1, :]; ty = tt_ref[1:2, :]; tz = tt_ref[2:3, :]
    dx = px - tx; dy = py - ty; dz = pz - tz
    loss = (dx * dx + dy * dy + dz * dz) * bm_ref[...]
    dist = ((tx - gat_ref[0:1, :]) * gat_ref[3:4, :]
            + (ty - gat_ref[1:2, :]) * gat_ref[4:5, :]
            + (tz - gat_ref[2:3, :]) * gat_ref[5:6, :])
    pen = jnp.where(dist > MCFG_EPS, 1.0, 0.0)
    total = jnp.sum(loss * pen, axis=1, keepdims=True)
    out_ref[...] = total * (1.0 / N_CLOTH)


def _loss_call(predT, targetT, bmask, gat):
    return pl.pallas_call(
        _loss_body,
        in_specs=[
            pl.BlockSpec((3, N_CLOTH), lambda: (0, 0)),
            pl.BlockSpec((3, N_CLOTH), lambda: (0, 0)),
            pl.BlockSpec((1, N_CLOTH), lambda: (0, 0)),
            pl.BlockSpec((6, N_CLOTH), lambda: (0, 0)),
        ],
        out_specs=pl.BlockSpec((1, 1), lambda: (0, 0)),
        out_shape=jax.ShapeDtypeStruct((1, 1), jnp.float32),
    )(predT, targetT, bmask, gat)


def kernel(cloth_pred_pos, cloth_target_pos, cloth_boundary_mask,
           obstacle_target_pos, obstacle_faces):
    faces = obstacle_faces.astype(jnp.int32)
    vx = obstacle_target_pos[:, 0]
    vy = obstacle_target_pos[:, 1]
    vz = obstacle_target_pos[:, 2]
    centf, normf = _faces_kernel_build()(
        vx, vy, vz, faces[0], faces[1], faces[2])

    nn_idx, nhat = _knn_call(
        cloth_target_pos,
        centf.reshape(4, N_OBS_FACES),
        normf.reshape(3, N_OBS_FACES))

    gatf = _gather_kernel_build()(
        centf, nhat.reshape(-1), nn_idx.reshape(-1))

    out = _loss_call(
        cloth_pred_pos.T, cloth_target_pos.T,
        cloth_boundary_mask.reshape(1, -1),
        gatf.reshape(6, N_CLOTH))
    return out[0, 0]
